# Initial kernel scaffold; baseline (speedup 1.0000x reference)
import jax
import jax.numpy as jnp
from jax.experimental import pallas as pl


def kernel(x, edge_index, W1, b1, W2, b2):
    raise NotImplementedError("write your pallas kernel here")


# speedup vs baseline: 36.8385x; 36.8385x over previous
import functools

import jax
import jax.numpy as jnp
from jax import lax
from jax.experimental import pallas as pl
from jax.experimental.pallas import tpu as pltpu
from jax.experimental.pallas import tpu_sc as plsc

N = 10000
D = 128
H = 64
C = 16
K = 10
ALPHA = 0.1

NT = 16
RPT = 632
NPAD = NT * RPT
CW = 128
NCH = 157
EPT = NCH * CW
EPAD = NT * EPT

_MESH = plsc.VectorSubcoreMesh(
    core_axis_name="c", subcore_axis_name="s", num_cores=1)
_SC_PARAMS = pltpu.CompilerParams(use_tc_tiling_on_sc=False)


def _zero_rows(ref, nrows):
    def body(i, _):
        ref[i, :] = jnp.zeros((C,), jnp.float32)
        return 0
    lax.fori_loop(0, nrows, body, 0)


def _deg_body(dst_hbm, deg_hbm, deg_sp, mydst, ones_v, rowbuf):
    t = lax.axis_index("s")
    base = t * RPT
    pltpu.sync_copy(dst_hbm.at[t], mydst)
    _zero_rows(rowbuf, RPT)
    pltpu.sync_copy(rowbuf, deg_sp.at[pl.ds(base, RPT)])

    def fill_ones(i, _):
        ones_v[i, :] = jnp.ones((C,), jnp.float32)
        return 0
    lax.fori_loop(0, CW, fill_ones, 0)
    plsc.subcore_barrier()

    def chunk(j, _):
        pltpu.sync_copy(ones_v, deg_sp.at[mydst.at[j]], add=True)
        return 0
    lax.fori_loop(0, NCH, chunk, 0)
    plsc.subcore_barrier()
    pltpu.sync_copy(deg_sp.at[pl.ds(base, RPT)], deg_hbm.at[pl.ds(base, RPT)])


_deg_call = pl.kernel(
    _deg_body,
    out_type=jax.ShapeDtypeStruct((NPAD, C), jnp.float32),
    mesh=_MESH,
    scratch_types=[
        pltpu.VMEM_SHARED((NPAD, C), jnp.float32),
        pltpu.VMEM((NCH, CW), jnp.int32),
        pltpu.VMEM((CW, C), jnp.float32),
        pltpu.VMEM((RPT, C), jnp.float32),
    ],
    compiler_params=_SC_PARAMS,
)


def _prop_body(src_hbm, dst_hbm, c1_hbm, c2_hbm, y0_hbm, yout_hbm,
               y_sp, s_sp, mysrc, mydst, gbuf, sbuf, ybuf, c1t, c2t, sem):
    t = lax.axis_index("s")
    base = t * RPT
    pltpu.sync_copy(src_hbm.at[t], mysrc)
    pltpu.sync_copy(dst_hbm.at[t], mydst)
    pltpu.sync_copy(c1_hbm.at[pl.ds(base, RPT)], c1t)
    pltpu.sync_copy(c2_hbm.at[pl.ds(base, RPT)], c2t)
    pltpu.sync_copy(y0_hbm.at[pl.ds(base, RPT)], ybuf)
    pltpu.sync_copy(ybuf, y_sp.at[pl.ds(base, RPT)])
    _zero_rows(sbuf, RPT)
    pltpu.sync_copy(sbuf, s_sp.at[pl.ds(base, RPT)])
    plsc.subcore_barrier()

    def round_body(_, carry):
        def chunk(j, c):
            pltpu.async_copy(y_sp.at[mysrc.at[j]], gbuf, sem).wait()
            pltpu.sync_copy(gbuf, s_sp.at[mydst.at[j]], add=True)
            return c
        lax.fori_loop(0, NCH, chunk, 0)
        plsc.subcore_barrier()
        pltpu.sync_copy(s_sp.at[pl.ds(base, RPT)], sbuf)

        def row(i, c):
            ybuf[i, :] = c1t[i, :] * (sbuf[i, :] + ybuf[i, :]) + c2t[i, :]
            sbuf[i, :] = jnp.zeros((C,), jnp.float32)
            return c
        lax.fori_loop(0, RPT, row, 0)
        pltpu.sync_copy(ybuf, y_sp.at[pl.ds(base, RPT)])
        pltpu.sync_copy(sbuf, s_sp.at[pl.ds(base, RPT)])
        plsc.subcore_barrier()
        return carry
    lax.fori_loop(0, K, round_body, 0)
    pltpu.sync_copy(ybuf, yout_hbm.at[pl.ds(base, RPT)])


_prop_call = pl.kernel(
    _prop_body,
    out_type=jax.ShapeDtypeStruct((NPAD, C), jnp.float32),
    mesh=_MESH,
    scratch_types=[
        pltpu.VMEM_SHARED((NPAD, C), jnp.float32),
        pltpu.VMEM_SHARED((NPAD, C), jnp.float32),
        pltpu.VMEM((NCH, CW), jnp.int32),
        pltpu.VMEM((NCH, CW), jnp.int32),
        pltpu.VMEM((CW, C), jnp.float32),
        pltpu.VMEM((RPT, C), jnp.float32),
        pltpu.VMEM((RPT, C), jnp.float32),
        pltpu.VMEM((RPT, C), jnp.float32),
        pltpu.VMEM((RPT, C), jnp.float32),
        pltpu.SemaphoreType.DMA,
    ],
    compiler_params=_SC_PARAMS,
)


def _prep_body(x_ref, w1_ref, b1_ref, w2_ref, b2_ref, deg16_ref,
               c1_ref, c2_ref, y0_ref, sdeg_ref):
    x = x_ref[...]
    h1 = jnp.maximum(
        jnp.dot(x, w1_ref[...], preferred_element_type=jnp.float32)
        + b1_ref[...], 0.0)
    h = (jnp.dot(h1, w2_ref[...], preferred_element_type=jnp.float32)
         + b2_ref[...])
    deg = deg16_ref[...][:, 0:1] + 1.0
    dinv = lax.rsqrt(deg)
    rows = lax.broadcasted_iota(jnp.int32, (NPAD, 1), 0)
    valid = rows < N
    y0 = jnp.where(valid, dinv * h, 0.0)
    c1 = jnp.where(valid, (1.0 - ALPHA) * dinv * dinv, 0.0)
    c1_ref[...] = jnp.broadcast_to(c1, (NPAD, C))
    c2_ref[...] = ALPHA * y0
    y0_ref[...] = y0
    sdeg_ref[...] = jnp.broadcast_to(jnp.sqrt(deg), (NPAD, C))


_prep_call = pl.pallas_call(
    _prep_body,
    out_shape=[
        jax.ShapeDtypeStruct((NPAD, C), jnp.float32),
        jax.ShapeDtypeStruct((NPAD, C), jnp.float32),
        jax.ShapeDtypeStruct((NPAD, C), jnp.float32),
        jax.ShapeDtypeStruct((NPAD, C), jnp.float32),
    ],
)


def _lsm_body(y_ref, sdeg_ref, out_ref):
    z = y_ref[...] * sdeg_ref[...]
    m = jnp.max(z, axis=1, keepdims=True)
    e = jnp.exp(z - m)
    out_ref[...] = z - m - jnp.log(jnp.sum(e, axis=1, keepdims=True))


_lsm_call = pl.pallas_call(
    _lsm_body,
    out_shape=jax.ShapeDtypeStruct((NPAD, C), jnp.float32),
)


def kernel(x, edge_index, W1, b1, W2, b2):
    src = edge_index[0].astype(jnp.int32)
    dst = edge_index[1].astype(jnp.int32)
    padv = jnp.full((EPAD - src.shape[0],), N, jnp.int32)
    src3 = jnp.concatenate([src, padv]).reshape(NT, NCH, CW)
    dst3 = jnp.concatenate([dst, padv]).reshape(NT, NCH, CW)
    xp = jnp.pad(x, ((0, NPAD - N), (0, 0)))

    deg16 = _deg_call(dst3)
    c1, c2, y0, sdeg = _prep_call(
        xp, W1, b1.reshape(1, H), W2, b2.reshape(1, C), deg16)
    y = _prop_call(src3, dst3, c1, c2, y0)
    out = _lsm_call(y, sdeg)
    return out[:N]


# speedup vs baseline: 49.8930x; 1.3544x over previous
import functools

import jax
import jax.numpy as jnp
from jax import lax
from jax.experimental import pallas as pl
from jax.experimental.pallas import tpu as pltpu
from jax.experimental.pallas import tpu_sc as plsc

N = 10000
D = 128
H = 64
C = 16
K = 10
ALPHA = 0.1

NT = 16
RPT = 632
NPAD = NT * RPT
CW = 128
NCH = 157
EPT = NCH * CW
EPAD = NT * EPT

_MESH = plsc.VectorSubcoreMesh(
    core_axis_name="c", subcore_axis_name="s", num_cores=1)
_SC_PARAMS = pltpu.CompilerParams(use_tc_tiling_on_sc=False)


def _zero_rows(ref, nrows):
    def body(i, _):
        ref[i, :] = jnp.zeros((C,), jnp.float32)
        return 0
    lax.fori_loop(0, nrows, body, 0)


def _deg_body(dst_hbm, deg_hbm, deg_sp, mydst, ones_v, rowbuf):
    t = lax.axis_index("s")
    base = t * RPT
    pltpu.sync_copy(dst_hbm.at[t], mydst)
    _zero_rows(rowbuf, RPT)
    pltpu.sync_copy(rowbuf, deg_sp.at[pl.ds(base, RPT)])

    def fill_ones(i, _):
        ones_v[i, :] = jnp.ones((C,), jnp.float32)
        return 0
    lax.fori_loop(0, CW, fill_ones, 0)
    plsc.subcore_barrier()

    def chunk(j, _):
        pltpu.sync_copy(ones_v, deg_sp.at[mydst.at[j]], add=True)
        return 0
    lax.fori_loop(0, NCH, chunk, 0)
    plsc.subcore_barrier()
    pltpu.sync_copy(deg_sp.at[pl.ds(base, RPT)], deg_hbm.at[pl.ds(base, RPT)])


_deg_call = pl.kernel(
    _deg_body,
    out_type=jax.ShapeDtypeStruct((NPAD, C), jnp.float32),
    mesh=_MESH,
    scratch_types=[
        pltpu.VMEM_SHARED((NPAD, C), jnp.float32),
        pltpu.VMEM((NCH, CW), jnp.int32),
        pltpu.VMEM((CW, C), jnp.float32),
        pltpu.VMEM((RPT, C), jnp.float32),
    ],
    compiler_params=_SC_PARAMS,
)


def _prop_body(src_hbm, dst_hbm, c1_hbm, c2_hbm, y0_hbm, yout_hbm,
               y_sp, s_sp, mysrc, mydst, gbuf, gbuf2, sbuf, ybuf, c1t, c2t,
               sem, sem2):
    t = lax.axis_index("s")
    base = t * RPT
    pltpu.sync_copy(src_hbm.at[t], mysrc)
    pltpu.sync_copy(dst_hbm.at[t], mydst)
    pltpu.sync_copy(c1_hbm.at[pl.ds(base, RPT)], c1t)
    pltpu.sync_copy(c2_hbm.at[pl.ds(base, RPT)], c2t)
    pltpu.sync_copy(y0_hbm.at[pl.ds(base, RPT)], ybuf)
    pltpu.sync_copy(ybuf, y_sp.at[pl.ds(base, RPT)])
    _zero_rows(sbuf, RPT)
    pltpu.sync_copy(sbuf, s_sp.at[pl.ds(base, RPT)])
    plsc.subcore_barrier()

    def round_body(_, carry):
        pltpu.async_copy(y_sp.at[mysrc.at[0]], gbuf, sem)

        def pair(i, c):
            j = 2 * i
            pltpu.make_async_copy(y_sp.at[mysrc.at[j]], gbuf, sem).wait()
            hb = pltpu.async_copy(y_sp.at[mysrc.at[j + 1]], gbuf2, sem2)
            pltpu.sync_copy(gbuf, s_sp.at[mydst.at[j]], add=True)
            pltpu.async_copy(y_sp.at[mysrc.at[j + 2]], gbuf, sem)
            hb.wait()
            pltpu.sync_copy(gbuf2, s_sp.at[mydst.at[j + 1]], add=True)
            return c
        lax.fori_loop(0, (NCH - 1) // 2, pair, 0)
        pltpu.make_async_copy(y_sp.at[mysrc.at[NCH - 1]], gbuf, sem).wait()
        pltpu.sync_copy(gbuf, s_sp.at[mydst.at[NCH - 1]], add=True)
        plsc.subcore_barrier()
        pltpu.sync_copy(s_sp.at[pl.ds(base, RPT)], sbuf)

        def row(i, c):
            ybuf[i, :] = c1t[i, :] * (sbuf[i, :] + ybuf[i, :]) + c2t[i, :]
            sbuf[i, :] = jnp.zeros((C,), jnp.float32)
            return c
        lax.fori_loop(0, RPT, row, 0)
        pltpu.sync_copy(ybuf, y_sp.at[pl.ds(base, RPT)])
        pltpu.sync_copy(sbuf, s_sp.at[pl.ds(base, RPT)])
        plsc.subcore_barrier()
        return carry
    lax.fori_loop(0, K, round_body, 0)
    pltpu.sync_copy(ybuf, yout_hbm.at[pl.ds(base, RPT)])


_prop_call = pl.kernel(
    _prop_body,
    out_type=jax.ShapeDtypeStruct((NPAD, C), jnp.float32),
    mesh=_MESH,
    scratch_types=[
        pltpu.VMEM_SHARED((NPAD, C), jnp.float32),
        pltpu.VMEM_SHARED((NPAD, C), jnp.float32),
        pltpu.VMEM((NCH, CW), jnp.int32),
        pltpu.VMEM((NCH, CW), jnp.int32),
        pltpu.VMEM((CW, C), jnp.float32),
        pltpu.VMEM((CW, C), jnp.float32),
        pltpu.VMEM((RPT, C), jnp.float32),
        pltpu.VMEM((RPT, C), jnp.float32),
        pltpu.VMEM((RPT, C), jnp.float32),
        pltpu.VMEM((RPT, C), jnp.float32),
        pltpu.SemaphoreType.DMA,
        pltpu.SemaphoreType.DMA,
    ],
    compiler_params=_SC_PARAMS,
)


def _prep_body(x_ref, w1_ref, b1_ref, w2_ref, b2_ref, deg16_ref,
               c1_ref, c2_ref, y0_ref, sdeg_ref):
    x = x_ref[...]
    h1 = jnp.maximum(
        jnp.dot(x, w1_ref[...], preferred_element_type=jnp.float32)
        + b1_ref[...], 0.0)
    h = (jnp.dot(h1, w2_ref[...], preferred_element_type=jnp.float32)
         + b2_ref[...])
    deg = deg16_ref[...][:, 0:1] + 1.0
    dinv = lax.rsqrt(deg)
    rows = lax.broadcasted_iota(jnp.int32, (NPAD, 1), 0)
    valid = rows < N
    y0 = jnp.where(valid, dinv * h, 0.0)
    c1 = jnp.where(valid, (1.0 - ALPHA) * dinv * dinv, 0.0)
    c1_ref[...] = jnp.broadcast_to(c1, (NPAD, C))
    c2_ref[...] = ALPHA * y0
    y0_ref[...] = y0
    sdeg_ref[...] = jnp.broadcast_to(jnp.sqrt(deg), (NPAD, C))


_prep_call = pl.pallas_call(
    _prep_body,
    out_shape=[
        jax.ShapeDtypeStruct((NPAD, C), jnp.float32),
        jax.ShapeDtypeStruct((NPAD, C), jnp.float32),
        jax.ShapeDtypeStruct((NPAD, C), jnp.float32),
        jax.ShapeDtypeStruct((NPAD, C), jnp.float32),
    ],
)


def _lsm_body(y_ref, sdeg_ref, out_ref):
    z = y_ref[...] * sdeg_ref[...]
    m = jnp.max(z, axis=1, keepdims=True)
    e = jnp.exp(z - m)
    out_ref[...] = z - m - jnp.log(jnp.sum(e, axis=1, keepdims=True))


_lsm_call = pl.pallas_call(
    _lsm_body,
    out_shape=jax.ShapeDtypeStruct((NPAD, C), jnp.float32),
)


def kernel(x, edge_index, W1, b1, W2, b2):
    src = edge_index[0].astype(jnp.int32)
    dst = edge_index[1].astype(jnp.int32)
    padv = jnp.full((EPAD - src.shape[0],), N, jnp.int32)
    src3 = jnp.concatenate([src, padv]).reshape(NT, NCH, CW)
    dst3 = jnp.concatenate([dst, padv]).reshape(NT, NCH, CW)
    xp = jnp.pad(x, ((0, NPAD - N), (0, 0)))

    deg16 = _deg_call(dst3)
    c1, c2, y0, sdeg = _prep_call(
        xp, W1, b1.reshape(1, H), W2, b2.reshape(1, C), deg16)
    y = _prop_call(src3, dst3, c1, c2, y0)
    out = _lsm_call(y, sdeg)
    return out[:N]


# speedup vs baseline: 51.3328x; 1.0289x over previous
import jax
import jax.numpy as jnp
from jax import lax
from jax.experimental import pallas as pl
from jax.experimental.pallas import tpu as pltpu
from jax.experimental.pallas import tpu_sc as plsc

N = 10000
D = 128
H = 64
C = 16
K = 10
ALPHA = 0.1

NT = 16
RPT = 632
NPAD = NT * RPT
CW = 128
NCH = 157
EPT = NCH * CW
EPAD = NT * EPT

_MESH = plsc.VectorSubcoreMesh(
    core_axis_name="c", subcore_axis_name="s", num_cores=1)
_SC_PARAMS = pltpu.CompilerParams(use_tc_tiling_on_sc=False)


def _rsqrt16(d):
    i = lax.bitcast_convert_type(d, jnp.int32)
    i = 0x5F3759DF - lax.shift_right_arithmetic(i, 1)
    r = lax.bitcast_convert_type(i, jnp.float32)
    for _ in range(3):
        r = r * (1.5 - 0.5 * d * r * r)
    return r


def _graph_body(src_hbm, dst_hbm, h_hbm, yout_hbm, sdeg_hbm,
                y_sp, s_sp, mysrc, mydst, gbuf, gbuf2, sbuf, ybuf,
                c1t, c2t, sdbuf, sem, sem2):
    t = lax.axis_index("s")
    base = t * RPT
    rows = pl.ds(base, RPT)
    pltpu.sync_copy(src_hbm.at[t], mysrc)
    pltpu.sync_copy(dst_hbm.at[t], mydst)
    pltpu.sync_copy(h_hbm.at[rows], ybuf)

    def zrow(i, c):
        sbuf[i, :] = jnp.zeros((C,), jnp.float32)
        return c
    lax.fori_loop(0, RPT, zrow, 0)
    pltpu.sync_copy(sbuf, s_sp.at[rows])

    def fill_ones(i, c):
        gbuf[i, :] = jnp.ones((C,), jnp.float32)
        return c
    lax.fori_loop(0, CW, fill_ones, 0)
    plsc.subcore_barrier()

    def deg_chunk(j, c):
        pltpu.sync_copy(gbuf, s_sp.at[mydst.at[j]], add=True)
        return c
    lax.fori_loop(0, NCH, deg_chunk, 0)
    plsc.subcore_barrier()

    pltpu.sync_copy(s_sp.at[rows], sbuf)

    def prep_row(i, c):
        d = sbuf[i, :] + 1.0
        dinv = _rsqrt16(d)
        c1t[i, :] = (1.0 - ALPHA) * dinv * dinv
        y0 = dinv * ybuf[i, :]
        ybuf[i, :] = y0
        c2t[i, :] = ALPHA * y0
        sdbuf[i, :] = d * dinv
        sbuf[i, :] = jnp.zeros((C,), jnp.float32)
        return c
    lax.fori_loop(0, RPT, prep_row, 0)
    pltpu.sync_copy(sdbuf, sdeg_hbm.at[rows])
    pltpu.sync_copy(ybuf, y_sp.at[rows])
    pltpu.sync_copy(sbuf, s_sp.at[rows])
    plsc.subcore_barrier()

    def round_body(_, carry):
        pltpu.async_copy(y_sp.at[mysrc.at[0]], gbuf, sem)

        def pair(i, c):
            j = 2 * i
            pltpu.make_async_copy(y_sp.at[mysrc.at[j]], gbuf, sem).wait()
            hb = pltpu.async_copy(y_sp.at[mysrc.at[j + 1]], gbuf2, sem2)
            pltpu.sync_copy(gbuf, s_sp.at[mydst.at[j]], add=True)
            pltpu.async_copy(y_sp.at[mysrc.at[j + 2]], gbuf, sem)
            hb.wait()
            pltpu.sync_copy(gbuf2, s_sp.at[mydst.at[j + 1]], add=True)
            return c
        lax.fori_loop(0, (NCH - 1) // 2, pair, 0)
        pltpu.make_async_copy(y_sp.at[mysrc.at[NCH - 1]], gbuf, sem).wait()
        pltpu.sync_copy(gbuf, s_sp.at[mydst.at[NCH - 1]], add=True)
        plsc.subcore_barrier()
        pltpu.sync_copy(s_sp.at[rows], sbuf)

        def row(i, c):
            ybuf[i, :] = c1t[i, :] * (sbuf[i, :] + ybuf[i, :]) + c2t[i, :]
            sbuf[i, :] = jnp.zeros((C,), jnp.float32)
            return c
        lax.fori_loop(0, RPT, row, 0)
        pltpu.sync_copy(ybuf, y_sp.at[rows])
        pltpu.sync_copy(sbuf, s_sp.at[rows])
        plsc.subcore_barrier()
        return carry
    lax.fori_loop(0, K, round_body, 0)
    pltpu.sync_copy(ybuf, yout_hbm.at[rows])


_graph_call = pl.kernel(
    _graph_body,
    out_type=(
        jax.ShapeDtypeStruct((NPAD, C), jnp.float32),
        jax.ShapeDtypeStruct((NPAD, C), jnp.float32),
    ),
    mesh=_MESH,
    scratch_types=[
        pltpu.VMEM_SHARED((NPAD, C), jnp.float32),
        pltpu.VMEM_SHARED((NPAD, C), jnp.float32),
        pltpu.VMEM((NCH, CW), jnp.int32),
        pltpu.VMEM((NCH, CW), jnp.int32),
        pltpu.VMEM((CW, C), jnp.float32),
        pltpu.VMEM((CW, C), jnp.float32),
        pltpu.VMEM((RPT, C), jnp.float32),
        pltpu.VMEM((RPT, C), jnp.float32),
        pltpu.VMEM((RPT, C), jnp.float32),
        pltpu.VMEM((RPT, C), jnp.float32),
        pltpu.VMEM((RPT, C), jnp.float32),
        pltpu.SemaphoreType.DMA,
        pltpu.SemaphoreType.DMA,
    ],
    compiler_params=_SC_PARAMS,
)


def _mlp_body(x_ref, w1_ref, b1_ref, w2_ref, b2_ref, h_ref):
    h1 = jnp.maximum(
        jnp.dot(x_ref[...], w1_ref[...], preferred_element_type=jnp.float32)
        + b1_ref[...], 0.0)
    h_ref[...] = (
        jnp.dot(h1, w2_ref[...], preferred_element_type=jnp.float32)
        + b2_ref[...])


_mlp_call = pl.pallas_call(
    _mlp_body,
    out_shape=jax.ShapeDtypeStruct((NPAD, C), jnp.float32),
)


def _lsm_body(y_ref, sdeg_ref, out_ref):
    z = y_ref[...] * sdeg_ref[...]
    m = jnp.max(z, axis=1, keepdims=True)
    e = jnp.exp(z - m)
    out_ref[...] = z - m - jnp.log(jnp.sum(e, axis=1, keepdims=True))


_lsm_call = pl.pallas_call(
    _lsm_body,
    out_shape=jax.ShapeDtypeStruct((NPAD, C), jnp.float32),
)


def kernel(x, edge_index, W1, b1, W2, b2):
    src = edge_index[0].astype(jnp.int32)
    dst = edge_index[1].astype(jnp.int32)
    padv = jnp.full((EPAD - src.shape[0],), N, jnp.int32)
    src3 = jnp.concatenate([src, padv]).reshape(NT, NCH, CW)
    dst3 = jnp.concatenate([dst, padv]).reshape(NT, NCH, CW)
    xp = jnp.pad(x, ((0, NPAD - N), (0, 0)))

    h = _mlp_call(xp, W1, b1.reshape(1, H), W2, b2.reshape(1, C))
    y, sdeg = _graph_call(src3, dst3, h)
    out = _lsm_call(y, sdeg)
    return out[:N]
